# baseline (speedup 1.0000x reference)
import jax
import jax.numpy as jnp
from jax.experimental import pallas as pl
from jax.experimental.pallas import tpu as pltpu

_N = 8192
_D = 32
_GRID = 8
_CHUNK = _N // _GRID


def _delta_dot_kernel(x3_ref, x4_ref, out_ref, acc_ref):
    i = pl.program_id(0)
    diff = x3_ref[...] - x4_ref[...]
    part = jnp.sum(diff, axis=0, keepdims=True)

    @pl.when(i == 0)
    def _init():
        acc_ref[...] = part

    @pl.when(i > 0)
    def _accum():
        acc_ref[...] = acc_ref[...] + part

    @pl.when(i == _GRID - 1)
    def _finish():
        s = acc_ref[...]
        scale = 2.0 / _N
        out_ref[...] = jnp.sum(s * s, axis=1, keepdims=True) * (scale * scale)


def kernel(x1, x2, x3, x4, W1, b1, Wq, bq, Wk, bk, s1Wl, s1bl, s1Wr,
           g1, be1, s2Wl, s2bl, s2Wr, g2, be2, W2, b2):
    out = pl.pallas_call(
        _delta_dot_kernel,
        grid=(_GRID,),
        in_specs=[
            pl.BlockSpec((_CHUNK, _D), lambda i: (i, 0)),
            pl.BlockSpec((_CHUNK, _D), lambda i: (i, 0)),
        ],
        out_specs=pl.BlockSpec((1, 1), lambda i: (0, 0)),
        out_shape=jax.ShapeDtypeStruct((1, 1), jnp.float32),
        scratch_shapes=[pltpu.VMEM((1, _D), jnp.float32)],
    )(x3, x4)
    return out[0, 0]


# speedup vs baseline: 1.1510x; 1.1510x over previous
import jax
import jax.numpy as jnp
from jax.experimental import pallas as pl


def _delta_dot_kernel(x3_ref, x4_ref, out_ref):
    diff = x3_ref[...] - x4_ref[...]
    col_sum = jnp.sum(diff, axis=0, keepdims=True)
    n = x3_ref.shape[0]
    scale = 2.0 / n
    val = jnp.sum(col_sum * col_sum, axis=1, keepdims=True)
    out_ref[...] = val * (scale * scale)


def kernel(x1, x2, x3, x4, W1, b1, Wq, bq, Wk, bk, s1Wl, s1bl, s1Wr,
           g1, be1, s2Wl, s2bl, s2Wr, g2, be2, W2, b2):
    out = pl.pallas_call(
        _delta_dot_kernel,
        out_shape=jax.ShapeDtypeStruct((1, 1), jnp.float32),
    )(x3, x4)
    return out[0, 0]
